# Initial kernel scaffold; baseline (speedup 1.0000x reference)
import jax
import jax.numpy as jnp
from jax.experimental import pallas as pl


def kernel(xcorr, nlag):
    raise NotImplementedError("write your pallas kernel here")


# speedup vs baseline: 25.4501x; 25.4501x over previous
import jax
import jax.numpy as jnp
from jax import lax
from jax.experimental import pallas as pl
from jax.experimental.pallas import tpu as pltpu
from jax.experimental.pallas import tpu_sc as plsc

NB, NCH, NX, NT = 16, 3, 64, 4096
L = 16
NCORES, NSUB = 2, 16
NWORK = NCORES * NSUB
NPAIR = NB * NX
PPW = NPAIR // NWORK
NCHUNK = NT // L
OFF = 8
NTP = NT + 2 * OFF
NEVAL = 201
NEC = 13


def _body(x_hbm, xs_hbm, scal_hbm,
          cc_hbm, w_hbm, st_hbm, si_hbm,
          rows0, rows1, rows2, xsv, scal, rcc, rw, rst, rsi):
    rows = (rows0, rows1, rows2)
    cid = lax.axis_index("c")
    sid = lax.axis_index("s")
    wid = sid * NCORES + cid

    pltpu.sync_copy(xs_hbm, xsv)
    pltpu.sync_copy(scal_hbm, scal)
    nlagf = jnp.max(scal[...])

    lanes = lax.iota(jnp.int32, L)

    sent_idx = jnp.where(lanes == 0, OFF - 1, OFF + NT)
    sent_msk = lanes < 2
    for c in range(NCH):
        plsc.store_scatter(rows[c], [sent_idx], jnp.zeros((L,), jnp.float32),
                           mask=sent_msk)

    def do_pair(pi, carry):
        p = wid * PPW + pi
        b = p // NX
        xi = p - b * NX
        rbase = b * (NCH * NX) + xi
        for c in range(NCH):
            pltpu.sync_copy(x_hbm.at[pl.ds((rbase + c * NX) * NT, NT)],
                            rows[c].at[pl.ds(OFF, NT)])

        zf = jnp.zeros((L,), jnp.float32)
        init = (zf, zf, lanes, zf, zf, lanes, zf, zf, lanes)

        def scan_body(j, st):
            o = OFF + j * L
            idxv = j * L + lanes
            out = []
            for c in range(NCH):
                b1, b2, iv = st[3 * c], st[3 * c + 1], st[3 * c + 2]
                ap = jnp.abs(rows[c][pl.ds(o - 1, L)])
                ac = jnp.abs(rows[c][pl.ds(o, L)])
                an = jnp.abs(rows[c][pl.ds(o + 1, L)])
                keep = ac >= jnp.maximum(ap, an)
                mv = jnp.where(keep, ac, 0.0)
                gt = mv > b1
                b2 = jnp.maximum(b2, jnp.minimum(mv, b1))
                b1 = jnp.maximum(b1, mv)
                iv = jnp.where(gt, idxv, iv)
                out += [b1, b2, iv]
            return tuple(out)

        st = lax.fori_loop(0, NCHUNK, scan_body, init, unroll=4)

        res = []
        for c in range(NCH):
            b1, b2, iv = st[3 * c], st[3 * c + 1], st[3 * c + 2]
            s0 = jnp.max(b1)
            i1 = jnp.min(jnp.where(b1 == s0, iv, jnp.int32(NT)))
            s1 = jnp.maximum(jnp.max(jnp.where(iv == i1, -1.0, b1)),
                             jnp.max(b2))
            im = jnp.maximum(i1 - 1, 0)
            ip = jnp.minimum(i1 + 1, NT - 1)
            gi = jnp.where(lanes == 0, im,
                           jnp.where(lanes == 1, i1, ip)) + OFF
            yv = jnp.abs(plsc.load_gather(rows[c], [gi]))
            y0 = jnp.max(jnp.where(lanes == 0, yv, -1.0))
            y1 = jnp.max(jnp.where(lanes == 1, yv, -1.0))
            y2 = jnp.max(jnp.where(lanes == 2, yv, -1.0))
            av = 0.5 * (y0 + y2) - y1
            bv = 0.5 * (y2 - y0)

            def ev_body(k, est):
                ym, kd = est
                t = xsv[pl.ds(k * L, L)]
                y = av * (t * t) + bv * t + y1
                gidx = k * L + lanes
                y = jnp.where(gidx <= NEVAL - 1, y, -jnp.inf)
                g = y > ym
                return (jnp.where(g, y, ym), jnp.where(g, gidx, kd))

            ym, kd = lax.fori_loop(0, NEC, ev_body,
                                   (jnp.full((L,), -jnp.inf, jnp.float32),
                                    lanes))
            yb = jnp.max(ym)
            ii = jnp.min(jnp.where(ym == yb, kd, jnp.int32(NEVAL)))
            sub = jnp.max(plsc.load_gather(xsv, [jnp.full((L,), ii, jnp.int32)]))
            wgt = (0.1 + 3.0 * (s0 - s1)) * (s0 * s0)
            tif = i1.astype(jnp.float32) + sub
            res.append((yb, wgt, tif))

        mc, mw, mt = res[0]
        for c in (1, 2):
            ccv, cw, ct = res[c]
            g = cw > mw
            mc = jnp.where(g, ccv, mc)
            mw = jnp.where(g, cw, mw)
            mt = jnp.where(g, ct, mt)
        siv = mt - nlagf
        stv = siv * jnp.float32(0.01)

        pidx = jnp.full((L,), pi, jnp.int32)
        m0 = lanes == 0
        plsc.store_scatter(rcc, [pidx], jnp.full((L,), mc, jnp.float32), mask=m0)
        plsc.store_scatter(rw, [pidx], jnp.full((L,), mw, jnp.float32), mask=m0)
        plsc.store_scatter(rst, [pidx], jnp.full((L,), stv, jnp.float32), mask=m0)
        plsc.store_scatter(rsi, [pidx], jnp.full((L,), siv, jnp.float32), mask=m0)
        return carry

    lax.fori_loop(0, PPW, do_pair, 0)

    base = wid * PPW
    pltpu.sync_copy(rcc, cc_hbm.at[pl.ds(base, PPW)])
    pltpu.sync_copy(rw, w_hbm.at[pl.ds(base, PPW)])
    pltpu.sync_copy(rst, st_hbm.at[pl.ds(base, PPW)])
    pltpu.sync_copy(rsi, si_hbm.at[pl.ds(base, PPW)])


_mesh = plsc.VectorSubcoreMesh(core_axis_name="c", subcore_axis_name="s",
                               num_cores=NCORES, num_subcores=NSUB)

_peaks = pl.kernel(
    _body,
    out_type=[jax.ShapeDtypeStruct((NPAIR,), jnp.float32)] * 4,
    mesh=_mesh,
    compiler_params=pltpu.CompilerParams(needs_layout_passes=False),
    scratch_types=[
        pltpu.VMEM((NTP,), jnp.float32),
        pltpu.VMEM((NTP,), jnp.float32),
        pltpu.VMEM((NTP,), jnp.float32),
        pltpu.VMEM((NEC * L,), jnp.float32),
        pltpu.VMEM((L,), jnp.float32),
        pltpu.VMEM((PPW,), jnp.float32),
        pltpu.VMEM((PPW,), jnp.float32),
        pltpu.VMEM((PPW,), jnp.float32),
        pltpu.VMEM((PPW,), jnp.float32),
    ],
)


def kernel(xcorr, nlag):
    x2 = xcorr.reshape(NB * NCH * NX * NT)
    xs = jnp.linspace(-1.0, 1.0, NEVAL, dtype=jnp.float32)
    xsp = jnp.concatenate([xs, jnp.zeros((NEC * L - NEVAL,), jnp.float32)])
    scal = jnp.full((L,), jnp.asarray(nlag, jnp.float32))
    cc, w, stt, si = _peaks(x2, xsp, scal)
    shp = (NB, 1, NX)
    return (cc.reshape(shp), w.reshape(shp), stt.reshape(shp), si.reshape(shp))


# speedup vs baseline: 38.0882x; 1.4966x over previous
import jax
import jax.numpy as jnp
from jax import lax
from jax.experimental import pallas as pl
from jax.experimental.pallas import tpu as pltpu
from jax.experimental.pallas import tpu_sc as plsc

NB, NCH, NX, NT = 16, 3, 64, 4096
L = 16
NCORES, NSUB = 2, 16
NWORK = NCORES * NSUB
NPAIR = NB * NX
PPW = NPAIR // NWORK
NCHUNK = NT // L
OFF = 8
NTP = NT + 2 * OFF
NEVAL = 201
NEC = 13


def _body(x_hbm, xs_hbm, scal_hbm,
          cc_hbm, w_hbm, st_hbm, si_hbm,
          a0, a1, a2, b0, b1r, b2r, xsv, scal,
          rcc, rw, rst, rsi, sem_a, sem_b):
    bufs_a = (a0, a1, a2)
    bufs_b = (b0, b1r, b2r)
    cid = lax.axis_index("c")
    sid = lax.axis_index("s")
    wid = sid * NCORES + cid

    pltpu.sync_copy(xs_hbm, xsv)
    pltpu.sync_copy(scal_hbm, scal)
    nlagf = jnp.max(scal[...])

    lanes = lax.iota(jnp.int32, L)

    sent_idx = jnp.where(lanes == 0, OFF - 1, OFF + NT)
    sent_msk = lanes < 2
    for bufs in (bufs_a, bufs_b):
        for c in range(NCH):
            plsc.store_scatter(bufs[c], [sent_idx],
                               jnp.zeros((L,), jnp.float32), mask=sent_msk)

    def row_addr(p, c):
        b = p // NX
        xi = p - b * NX
        return (b * (NCH * NX) + c * NX + xi) * NT

    def issue(bufs, sem, p):
        for c in range(NCH):
            pltpu.async_copy(x_hbm.at[pl.ds(row_addr(p, c), NT)],
                             bufs[c].at[pl.ds(OFF, NT)], sem)

    def drain(bufs, sem):
        for c in range(NCH):
            pltpu.make_async_copy(x_hbm.at[pl.ds(0, NT)],
                                  bufs[c].at[pl.ds(OFF, NT)], sem).wait()

    def compute(bufs, pi):
        zf = jnp.zeros((L,), jnp.float32)
        init = (zf, zf, lanes, zf, zf, lanes, zf, zf, lanes)

        def scan_body(j, st):
            o = OFF + j * L
            idxv = j * L + lanes
            out = []
            for c in range(NCH):
                b1, b2, iv = st[3 * c], st[3 * c + 1], st[3 * c + 2]
                ap = jnp.abs(bufs[c][pl.ds(o - 1, L)])
                ac = jnp.abs(bufs[c][pl.ds(o, L)])
                an = jnp.abs(bufs[c][pl.ds(o + 1, L)])
                keep = ac >= jnp.maximum(ap, an)
                mv = jnp.where(keep, ac, 0.0)
                gt = mv > b1
                b2 = jnp.maximum(b2, jnp.minimum(mv, b1))
                b1 = jnp.maximum(b1, mv)
                iv = jnp.where(gt, idxv, iv)
                out += [b1, b2, iv]
            return tuple(out)

        st = lax.fori_loop(0, NCHUNK, scan_body, init, unroll=4)

        res = []
        for c in range(NCH):
            b1, b2, iv = st[3 * c], st[3 * c + 1], st[3 * c + 2]
            s0 = jnp.max(b1)
            i1 = jnp.min(jnp.where(b1 == s0, iv, jnp.int32(NT)))
            s1 = jnp.maximum(jnp.max(jnp.where(iv == i1, -1.0, b1)),
                             jnp.max(b2))
            im = jnp.maximum(i1 - 1, 0)
            ip = jnp.minimum(i1 + 1, NT - 1)
            gi = jnp.where(lanes == 0, im,
                           jnp.where(lanes == 1, i1, ip)) + OFF
            yv = jnp.abs(plsc.load_gather(bufs[c], [gi]))
            y0 = jnp.max(jnp.where(lanes == 0, yv, -1.0))
            y1 = jnp.max(jnp.where(lanes == 1, yv, -1.0))
            y2 = jnp.max(jnp.where(lanes == 2, yv, -1.0))
            av = 0.5 * (y0 + y2) - y1
            bv = 0.5 * (y2 - y0)

            def ev_body(k, est):
                ym, kd = est
                t = xsv[pl.ds(k * L, L)]
                y = av * (t * t) + bv * t + y1
                gidx = k * L + lanes
                y = jnp.where(gidx <= NEVAL - 1, y, -jnp.inf)
                g = y > ym
                return (jnp.where(g, y, ym), jnp.where(g, gidx, kd))

            ym, kd = lax.fori_loop(0, NEC, ev_body,
                                   (jnp.full((L,), -jnp.inf, jnp.float32),
                                    lanes))
            yb = jnp.max(ym)
            ii = jnp.min(jnp.where(ym == yb, kd, jnp.int32(NEVAL)))
            sub = jnp.max(plsc.load_gather(xsv, [jnp.full((L,), ii, jnp.int32)]))
            wgt = (0.1 + 3.0 * (s0 - s1)) * (s0 * s0)
            tif = i1.astype(jnp.float32) + sub
            res.append((yb, wgt, tif))

        mc, mw, mt = res[0]
        for c in (1, 2):
            ccv, cw, ct = res[c]
            g = cw > mw
            mc = jnp.where(g, ccv, mc)
            mw = jnp.where(g, cw, mw)
            mt = jnp.where(g, ct, mt)
        siv = mt - nlagf
        stv = siv * jnp.float32(0.01)

        pidx = jnp.full((L,), pi, jnp.int32)
        m0 = lanes == 0
        plsc.store_scatter(rcc, [pidx], jnp.full((L,), mc, jnp.float32), mask=m0)
        plsc.store_scatter(rw, [pidx], jnp.full((L,), mw, jnp.float32), mask=m0)
        plsc.store_scatter(rst, [pidx], jnp.full((L,), stv, jnp.float32), mask=m0)
        plsc.store_scatter(rsi, [pidx], jnp.full((L,), siv, jnp.float32), mask=m0)

    pbase = wid * PPW
    issue(bufs_a, sem_a, pbase)

    def half(i, carry):
        p_a = pbase + 2 * i
        drain(bufs_a, sem_a)
        issue(bufs_b, sem_b, p_a + 1)
        compute(bufs_a, 2 * i)
        drain(bufs_b, sem_b)

        @pl.when(i < PPW // 2 - 1)
        def _():
            issue(bufs_a, sem_a, p_a + 2)

        compute(bufs_b, 2 * i + 1)
        return carry

    lax.fori_loop(0, PPW // 2, half, 0)

    base = wid * PPW
    pltpu.sync_copy(rcc, cc_hbm.at[pl.ds(base, PPW)])
    pltpu.sync_copy(rw, w_hbm.at[pl.ds(base, PPW)])
    pltpu.sync_copy(rst, st_hbm.at[pl.ds(base, PPW)])
    pltpu.sync_copy(rsi, si_hbm.at[pl.ds(base, PPW)])


_mesh = plsc.VectorSubcoreMesh(core_axis_name="c", subcore_axis_name="s",
                               num_cores=NCORES, num_subcores=NSUB)

_peaks = pl.kernel(
    _body,
    out_type=[jax.ShapeDtypeStruct((NPAIR,), jnp.float32)] * 4,
    mesh=_mesh,
    compiler_params=pltpu.CompilerParams(needs_layout_passes=False),
    scratch_types=[
        pltpu.VMEM((NTP,), jnp.float32),
        pltpu.VMEM((NTP,), jnp.float32),
        pltpu.VMEM((NTP,), jnp.float32),
        pltpu.VMEM((NTP,), jnp.float32),
        pltpu.VMEM((NTP,), jnp.float32),
        pltpu.VMEM((NTP,), jnp.float32),
        pltpu.VMEM((NEC * L,), jnp.float32),
        pltpu.VMEM((L,), jnp.float32),
        pltpu.VMEM((PPW,), jnp.float32),
        pltpu.VMEM((PPW,), jnp.float32),
        pltpu.VMEM((PPW,), jnp.float32),
        pltpu.VMEM((PPW,), jnp.float32),
        pltpu.SemaphoreType.DMA,
        pltpu.SemaphoreType.DMA,
    ],
)


def kernel(xcorr, nlag):
    x2 = xcorr.reshape(NB * NCH * NX * NT)
    xs = jnp.linspace(-1.0, 1.0, NEVAL, dtype=jnp.float32)
    xsp = jnp.concatenate([xs, jnp.zeros((NEC * L - NEVAL,), jnp.float32)])
    scal = jnp.full((L,), jnp.asarray(nlag, jnp.float32))
    cc, w, stt, si = _peaks(x2, xsp, scal)
    shp = (NB, 1, NX)
    return (cc.reshape(shp), w.reshape(shp), stt.reshape(shp), si.reshape(shp))


# speedup vs baseline: 38.7603x; 1.0176x over previous
import jax
import jax.numpy as jnp
from jax import lax
from jax.experimental import pallas as pl
from jax.experimental.pallas import tpu as pltpu
from jax.experimental.pallas import tpu_sc as plsc

NB, NCH, NX, NT = 16, 3, 64, 4096
L = 16
NCORES, NSUB = 2, 16
NWORK = NCORES * NSUB
NPAIR = NB * NX
PPW = NPAIR // NWORK
NCHUNK = NT // L
OFF = 8
NTP = NT + 2 * OFF
NEVAL = 201
NEC = 13


def _body(x_hbm, xs_hbm, scal_hbm,
          cc_hbm, w_hbm, st_hbm, si_hbm,
          a0, a1, a2, b0, b1r, b2r, xsv, scal,
          rcc, rw, rst, rsi, sem_a, sem_b):
    bufs_a = (a0, a1, a2)
    bufs_b = (b0, b1r, b2r)
    cid = lax.axis_index("c")
    sid = lax.axis_index("s")
    wid = sid * NCORES + cid

    pltpu.sync_copy(xs_hbm, xsv)
    pltpu.sync_copy(scal_hbm, scal)
    nlagf = jnp.max(scal[...])

    lanes = lax.iota(jnp.int32, L)

    sent_idx = jnp.where(lanes == 0, OFF - 1, OFF + NT)
    sent_msk = lanes < 2
    for bufs in (bufs_a, bufs_b):
        for c in range(NCH):
            plsc.store_scatter(bufs[c], [sent_idx],
                               jnp.zeros((L,), jnp.float32), mask=sent_msk)

    def row_addr(p, c):
        b = p // NX
        xi = p - b * NX
        return (b * (NCH * NX) + c * NX + xi) * NT

    def issue(bufs, sem, p):
        for c in range(NCH):
            pltpu.async_copy(x_hbm.at[pl.ds(row_addr(p, c), NT)],
                             bufs[c].at[pl.ds(OFF, NT)], sem)

    def drain(bufs, sem):
        for c in range(NCH):
            pltpu.make_async_copy(x_hbm.at[pl.ds(0, NT)],
                                  bufs[c].at[pl.ds(OFF, NT)], sem).wait()

    def compute(bufs, pi):
        zf = jnp.zeros((L,), jnp.float32)
        init = (zf, zf, lanes, zf, zf, lanes, zf, zf, lanes)

        def scan_body(j, st):
            o = OFF + j * L
            idxv = j * L + lanes
            out = []
            for c in range(NCH):
                b1, b2, iv = st[3 * c], st[3 * c + 1], st[3 * c + 2]
                ap = jnp.abs(bufs[c][pl.ds(o - 1, L)])
                ac = jnp.abs(bufs[c][pl.ds(o, L)])
                an = jnp.abs(bufs[c][pl.ds(o + 1, L)])
                keep = ac >= jnp.maximum(ap, an)
                mv = jnp.where(keep, ac, 0.0)
                gt = mv > b1
                b2 = jnp.maximum(b2, jnp.minimum(mv, b1))
                b1 = jnp.maximum(b1, mv)
                iv = jnp.where(gt, idxv, iv)
                out += [b1, b2, iv]
            return tuple(out)

        st = plsc.parallel_loop(0, NCHUNK, unroll=8, carry=init)(
            lambda j, c: scan_body(j, c))

        coef = []
        for c in range(NCH):
            b1, b2, iv = st[3 * c], st[3 * c + 1], st[3 * c + 2]
            s0 = jnp.max(b1)
            i1 = jnp.min(jnp.where(b1 == s0, iv, jnp.int32(NT)))
            s1 = jnp.maximum(jnp.max(jnp.where(iv == i1, -1.0, b1)),
                             jnp.max(b2))
            im = jnp.maximum(i1 - 1, 0)
            ip = jnp.minimum(i1 + 1, NT - 1)
            gi = jnp.where(lanes == 0, im,
                           jnp.where(lanes == 1, i1, ip)) + OFF
            yv = jnp.abs(plsc.load_gather(bufs[c], [gi]))
            y0 = jnp.max(jnp.where(lanes == 0, yv, -1.0))
            y1 = jnp.max(jnp.where(lanes == 1, yv, -1.0))
            y2 = jnp.max(jnp.where(lanes == 2, yv, -1.0))
            av = 0.5 * (y0 + y2) - y1
            bv = 0.5 * (y2 - y0)
            coef.append((s0, s1, i1, av, bv, y1))

        neg = jnp.full((L,), -jnp.inf, jnp.float32)
        ev_init = (neg, lanes, neg, lanes, neg, lanes)

        def ev_body(k, est):
            t = xsv[pl.ds(k * L, L)]
            t2 = t * t
            gidx = k * L + lanes
            pad_ok = gidx <= NEVAL - 1
            out = []
            for c in range(NCH):
                ym, kd = est[2 * c], est[2 * c + 1]
                av, bv, y1 = coef[c][3], coef[c][4], coef[c][5]
                y = av * t2 + bv * t + y1
                y = jnp.where(pad_ok, y, -jnp.inf)
                g = y > ym
                out += [jnp.where(g, y, ym), jnp.where(g, gidx, kd)]
            return tuple(out)

        est = lax.fori_loop(0, NEC, ev_body, ev_init)
        res = []
        for c in range(NCH):
            s0, s1, i1 = coef[c][0], coef[c][1], coef[c][2]
            ym, kd = est[2 * c], est[2 * c + 1]
            yb = jnp.max(ym)
            ii = jnp.min(jnp.where(ym == yb, kd, jnp.int32(NEVAL)))
            sub = jnp.max(plsc.load_gather(xsv, [jnp.full((L,), ii, jnp.int32)]))
            wgt = (0.1 + 3.0 * (s0 - s1)) * (s0 * s0)
            tif = i1.astype(jnp.float32) + sub
            res.append((yb, wgt, tif))

        mc, mw, mt = res[0]
        for c in (1, 2):
            ccv, cw, ct = res[c]
            g = cw > mw
            mc = jnp.where(g, ccv, mc)
            mw = jnp.where(g, cw, mw)
            mt = jnp.where(g, ct, mt)
        siv = mt - nlagf
        stv = siv * jnp.float32(0.01)

        pidx = jnp.full((L,), pi, jnp.int32)
        m0 = lanes == 0
        plsc.store_scatter(rcc, [pidx], jnp.full((L,), mc, jnp.float32), mask=m0)
        plsc.store_scatter(rw, [pidx], jnp.full((L,), mw, jnp.float32), mask=m0)
        plsc.store_scatter(rst, [pidx], jnp.full((L,), stv, jnp.float32), mask=m0)
        plsc.store_scatter(rsi, [pidx], jnp.full((L,), siv, jnp.float32), mask=m0)

    pbase = wid * PPW
    issue(bufs_a, sem_a, pbase)

    def half(i, carry):
        p_a = pbase + 2 * i
        drain(bufs_a, sem_a)
        issue(bufs_b, sem_b, p_a + 1)
        compute(bufs_a, 2 * i)
        drain(bufs_b, sem_b)

        @pl.when(i < PPW // 2 - 1)
        def _():
            issue(bufs_a, sem_a, p_a + 2)

        compute(bufs_b, 2 * i + 1)
        return carry

    lax.fori_loop(0, PPW // 2, half, 0)

    base = wid * PPW
    pltpu.sync_copy(rcc, cc_hbm.at[pl.ds(base, PPW)])
    pltpu.sync_copy(rw, w_hbm.at[pl.ds(base, PPW)])
    pltpu.sync_copy(rst, st_hbm.at[pl.ds(base, PPW)])
    pltpu.sync_copy(rsi, si_hbm.at[pl.ds(base, PPW)])


_mesh = plsc.VectorSubcoreMesh(core_axis_name="c", subcore_axis_name="s",
                               num_cores=NCORES, num_subcores=NSUB)

_peaks = pl.kernel(
    _body,
    out_type=[jax.ShapeDtypeStruct((NPAIR,), jnp.float32)] * 4,
    mesh=_mesh,
    compiler_params=pltpu.CompilerParams(needs_layout_passes=False),
    scratch_types=[
        pltpu.VMEM((NTP,), jnp.float32),
        pltpu.VMEM((NTP,), jnp.float32),
        pltpu.VMEM((NTP,), jnp.float32),
        pltpu.VMEM((NTP,), jnp.float32),
        pltpu.VMEM((NTP,), jnp.float32),
        pltpu.VMEM((NTP,), jnp.float32),
        pltpu.VMEM((NEC * L,), jnp.float32),
        pltpu.VMEM((L,), jnp.float32),
        pltpu.VMEM((PPW,), jnp.float32),
        pltpu.VMEM((PPW,), jnp.float32),
        pltpu.VMEM((PPW,), jnp.float32),
        pltpu.VMEM((PPW,), jnp.float32),
        pltpu.SemaphoreType.DMA,
        pltpu.SemaphoreType.DMA,
    ],
)


def kernel(xcorr, nlag):
    x2 = xcorr.reshape(NB * NCH * NX * NT)
    xs = jnp.linspace(-1.0, 1.0, NEVAL, dtype=jnp.float32)
    xsp = jnp.concatenate([xs, jnp.zeros((NEC * L - NEVAL,), jnp.float32)])
    scal = jnp.full((L,), jnp.asarray(nlag, jnp.float32))
    cc, w, stt, si = _peaks(x2, xsp, scal)
    shp = (NB, 1, NX)
    return (cc.reshape(shp), w.reshape(shp), stt.reshape(shp), si.reshape(shp))
